# baseline (speedup 1.0000x reference)
import functools

import jax
import jax.numpy as jnp
from jax import lax
from jax.experimental import pallas as pl
from jax.experimental.pallas import tpu as pltpu
from jax.experimental.pallas import tpu_sc as plsc

N = 10000
E = 160000
NUM_CLASSES = 6890

NC, NS, LANES = 2, 16, 16
NW = NC * NS
EPAD = 163840
EW = EPAD // NW
NP = 10240
NROWS = NP // NS

_MESH = plsc.VectorSubcoreMesh(core_axis_name="c", subcore_axis_name="s",
                               num_cores=NC, num_subcores=NS)

_CORNERS = [(b0, b1, b2) for b0 in (0, 1) for b1 in (0, 1) for b2 in (0, 1)]


def _zero_vmem(zb):
    rows, cols = zb.shape
    z16 = jnp.zeros((16,), jnp.float32)
    for i in range(rows):
        for j in range(cols // 16):
            zb[i, pl.ds(j * 16, 16)] = z16


def _zero_spmem(zb, dst, row0):
    zr = zb.shape[0]
    for z in range(NROWS // zr):
        pltpu.sync_copy(zb, dst.at[pl.ds(row0 + z * zr, zr)])


def _corner_weights(p0, p1, p2, mask):
    pos0, pos1, pos2 = p0 * 2.0, p1 * 2.0, p2 * 2.0
    one = jnp.float32(1.0)
    lo0 = jnp.where(pos0 >= one, one, 0.0)
    lo1 = jnp.where(pos1 >= one, one, 0.0)
    lo2 = jnp.where(pos2 >= one, one, 0.0)
    f0, f1, f2 = pos0 - lo0, pos1 - lo1, pos2 - lo2
    a0, a1, a2 = one - f0, one - f1, one - f2
    kb = (lo0.astype(jnp.int32) * 9 + lo1.astype(jnp.int32) * 3
          + lo2.astype(jnp.int32))
    ws, kcs = [], []
    for b0, b1, b2 in _CORNERS:
        w = (f0 if b0 else a0) * (f1 if b1 else a1) * (f2 if b2 else a2)
        ws.append(jnp.where(mask, w, 0.0))
        kcs.append(kb + (b0 * 9 + b1 * 3 + b2))
    return ws, kcs


def _k0_body(src_hbm, dst_hbm, ps_hbm, x_hbm,
             w8_hbm, gidx_hbm, s1_hbm,
             x_v, zb_v, src_v, dst_v, ps_v, w8_v, gidx_v, rows_v, s1_sp):
    cid = lax.axis_index("c")
    sid = lax.axis_index("s")
    wid = cid * NS + sid
    base = wid * EW

    pltpu.sync_copy(x_hbm, x_v)
    _zero_vmem(zb_v)
    _zero_vmem(rows_v)
    _zero_spmem(zb_v, s1_sp, sid * NROWS)
    plsc.subcore_barrier()

    lanes = lax.iota(jnp.int32, 16)
    deg27 = jnp.full((16,), 27, jnp.int32)
    z16 = jnp.zeros((16,), jnp.float32)

    def chunk(i, carry):
        off = base + i * 128
        pltpu.sync_copy(src_hbm.at[pl.ds(off, 128)], src_v)
        pltpu.sync_copy(dst_hbm.at[pl.ds(off, 128)], dst_v.at[0])
        for d in range(3):
            pltpu.sync_copy(ps_hbm.at[pl.ds(d * EPAD + off, 128)],
                            ps_v.at[d])
        for r in range(128):
            for j in range(2):
                rows_v[r, pl.ds(j * 16, 16)] = z16
        for g in range(8):
            sl = pl.ds(g * 16, 16)
            sv = src_v[sl]
            mask = sv >= 0
            svs = jnp.maximum(sv, 0)
            xv = plsc.load_gather(x_v, [svs])
            ws, kcs = _corner_weights(ps_v[0, sl], ps_v[1, sl], ps_v[2, sl],
                                      mask)
            rowsel = g * 16 + lanes
            for ci in range(8):
                w8_v[ci, sl] = ws[ci]
                gidx_v[ci, sl] = svs * 27 + kcs[ci]
                plsc.addupdate_scatter(rows_v, [rowsel, kcs[ci]], ws[ci] * xv)
            plsc.store_scatter(rows_v, [rowsel, deg27],
                               jnp.where(mask, 1.0, 0.0))
        for ci in range(8):
            pltpu.sync_copy(w8_v.at[ci],
                            w8_hbm.at[pl.ds(ci * EPAD + off, 128)])
            pltpu.sync_copy(gidx_v.at[ci],
                            gidx_hbm.at[pl.ds(ci * EPAD + off, 128)])
        pltpu.sync_copy(rows_v, s1_sp.at[dst_v.at[0]], add=True)
        return carry

    lax.fori_loop(0, EW // 128, chunk, 0)

    plsc.subcore_barrier()
    row0 = sid * NROWS
    for z in range(5):
        pltpu.sync_copy(s1_sp.at[pl.ds(row0 + z * 128, 128)],
                        s1_hbm.at[cid, pl.ds(row0 + z * 128, 128)])


@jax.jit
def _k0(srcp, dstp, psT, xflat):
    f = pl.kernel(
        _k0_body,
        out_type=[
            jax.ShapeDtypeStruct((8 * EPAD,), jnp.float32),
            jax.ShapeDtypeStruct((8 * EPAD,), jnp.int32),
            jax.ShapeDtypeStruct((NC, NP, 128), jnp.float32),
        ],
        mesh=_MESH,
        compiler_params=pltpu.CompilerParams(needs_layout_passes=False),
        scratch_types=[
            pltpu.VMEM((NP,), jnp.float32),
            pltpu.VMEM((64, 128), jnp.float32),
            pltpu.VMEM((128,), jnp.int32),
            pltpu.VMEM((1, 128), jnp.int32),
            pltpu.VMEM((3, 128), jnp.float32),
            pltpu.VMEM((8, 128), jnp.float32),
            pltpu.VMEM((8, 128), jnp.int32),
            pltpu.VMEM((128, 128), jnp.float32),
            pltpu.VMEM_SHARED((NP, 128), jnp.float32),
        ],
    )
    return f(srcp, dstp, psT, xflat)


CS = 16


def _make_k23_body(out_dim, paired):
    G = 4 if paired else 8
    NR = G * CS

    def body(tab_hbm, gidx_hbm, w8_hbm, dst_hbm,
             agg_hbm,
             zb_v, idx_v, w8f_v, dst_v, rows_v, out_v, agg_sp):
        cid = lax.axis_index("c")
        sid = lax.axis_index("s")
        wid = cid * NS + sid
        base = wid * EW

        _zero_vmem(zb_v)
        _zero_spmem(zb_v, agg_sp, sid * NROWS)
        plsc.subcore_barrier()
        z16 = jnp.zeros((16,), jnp.float32)
        if paired:
            for e in range(CS):
                for j in range(4, 8):
                    out_v[e, pl.ds(j * 16, 16)] = z16

        def chunk(i, carry):
            off = base + i * CS
            pltpu.sync_copy(dst_hbm.at[pl.ds(off, CS)], dst_v.at[0])
            for ci in range(8):
                pltpu.sync_copy(w8_hbm.at[pl.ds(ci * EPAD + off, CS)],
                                w8f_v.at[pl.ds(ci * CS, CS)])
            for gi in range(G):
                ci = 2 * gi if paired else gi
                pltpu.sync_copy(gidx_hbm.at[pl.ds(ci * EPAD + off, CS)],
                                idx_v.at[0, pl.ds(gi * CS, CS)])
            pltpu.sync_copy(tab_hbm.at[idx_v.at[0]], rows_v)

            def edge(e, c2):
                wsp = [plsc.load_gather(
                    w8f_v, [jnp.full((16,), ci * CS, jnp.int32) + e])
                    for ci in range(8)]
                for oc in range(out_dim // 16):
                    osl = pl.ds(oc * 16, 16)
                    if paired:
                        acc = wsp[0] * rows_v[e, osl]
                        acc = acc + wsp[1] * rows_v[e, pl.ds(64 + oc * 16, 16)]
                        for gi in range(1, 4):
                            r = gi * CS + e
                            acc = acc + wsp[2 * gi] * rows_v[r, osl]
                            acc = acc + (wsp[2 * gi + 1]
                                         * rows_v[r, pl.ds(64 + oc * 16, 16)])
                    else:
                        acc = wsp[0] * rows_v[e, osl]
                        for ci in range(1, 8):
                            acc = acc + wsp[ci] * rows_v[ci * CS + e, osl]
                    out_v[e, osl] = acc
                return c2

            lax.fori_loop(0, CS, edge, 0)
            pltpu.sync_copy(out_v, agg_sp.at[dst_v.at[0]], add=True)
            return carry

        lax.fori_loop(0, EW // CS, chunk, 0)

        plsc.subcore_barrier()
        row0 = sid * NROWS
        for z in range(5):
            pltpu.sync_copy(agg_sp.at[pl.ds(row0 + z * 128, 128)],
                            agg_hbm.at[cid, pl.ds(row0 + z * 128, 128)])

    return body


@functools.partial(jax.jit, static_argnames=("out_dim", "paired"))
def _k23(tab, gidx, w8, dstp, out_dim, paired):
    nr = (4 if paired else 8) * CS
    f = pl.kernel(
        _make_k23_body(out_dim, paired),
        out_type=jax.ShapeDtypeStruct((NC, NP, 128), jnp.float32),
        mesh=_MESH,
        compiler_params=pltpu.CompilerParams(needs_layout_passes=False),
        scratch_types=[
            pltpu.VMEM((64, 128), jnp.float32),
            pltpu.VMEM((1, nr), jnp.int32),
            pltpu.VMEM((8 * CS,), jnp.float32),
            pltpu.VMEM((1, CS), jnp.int32),
            pltpu.VMEM((nr, 128), jnp.float32),
            pltpu.VMEM((CS, 128), jnp.float32),
            pltpu.VMEM_SHARED((NP, 128), jnp.float32),
        ],
    )
    return f(tab, gidx, w8, dstp)


BM = 512


def _tck1_body(s1a_ref, s1b_ref, x_ref, w1f_ref, r1_ref, b1_ref, w2f_ref,
               h1_ref, recip_ref, tab2_ref):
    s = s1a_ref[...] + s1b_ref[...]
    deg = s[:, 27:28]
    recip = 1.0 / jnp.maximum(deg, 1.0)
    agg = jnp.dot(s, w1f_ref[...], preferred_element_type=jnp.float32) * recip
    h1 = jax.nn.relu(agg + x_ref[...] * r1_ref[...] + b1_ref[...])
    h1_ref[...] = h1
    recip_ref[...] = recip
    tab2_ref[...] = jnp.dot(h1, w2f_ref[...],
                            preferred_element_type=jnp.float32)


@jax.jit
def _tck1(s1a, s1b, x, w1f, r1, b1, w2f):
    grid = (NP // BM,)
    return pl.pallas_call(
        _tck1_body,
        grid=grid,
        in_specs=[
            pl.BlockSpec((BM, 128), lambda i: (i, 0)),
            pl.BlockSpec((BM, 128), lambda i: (i, 0)),
            pl.BlockSpec((BM, 1), lambda i: (i, 0)),
            pl.BlockSpec((128, 32), lambda i: (0, 0)),
            pl.BlockSpec((1, 32), lambda i: (0, 0)),
            pl.BlockSpec((1, 32), lambda i: (0, 0)),
            pl.BlockSpec((32, 27 * 128), lambda i: (0, 0)),
        ],
        out_specs=[
            pl.BlockSpec((BM, 32), lambda i: (i, 0)),
            pl.BlockSpec((BM, 1), lambda i: (i, 0)),
            pl.BlockSpec((BM, 27 * 128), lambda i: (i, 0)),
        ],
        out_shape=[
            jax.ShapeDtypeStruct((NP, 32), jnp.float32),
            jax.ShapeDtypeStruct((NP, 1), jnp.float32),
            jax.ShapeDtypeStruct((NP, 27 * 128), jnp.float32),
        ],
    )(s1a, s1b, x, w1f, r1, b1, w2f)


def _tck2_body(a2a_ref, a2b_ref, recip_ref, h1_ref, r2_ref, b2_ref, w3f_ref,
               h2_ref, tab3_ref):
    agg = (a2a_ref[:, :64] + a2b_ref[:, :64]) * recip_ref[...]
    h2 = jax.nn.relu(agg + jnp.dot(h1_ref[...], r2_ref[...],
                                   preferred_element_type=jnp.float32)
                     + b2_ref[...])
    h2_ref[...] = h2
    tab3_ref[...] = jnp.dot(h2, w3f_ref[...],
                            preferred_element_type=jnp.float32)


@jax.jit
def _tck2(a2a, a2b, recip, h1, r2, b2, w3f):
    grid = (NP // BM,)
    return pl.pallas_call(
        _tck2_body,
        grid=grid,
        in_specs=[
            pl.BlockSpec((BM, 128), lambda i: (i, 0)),
            pl.BlockSpec((BM, 128), lambda i: (i, 0)),
            pl.BlockSpec((BM, 1), lambda i: (i, 0)),
            pl.BlockSpec((BM, 32), lambda i: (i, 0)),
            pl.BlockSpec((32, 64), lambda i: (0, 0)),
            pl.BlockSpec((1, 64), lambda i: (0, 0)),
            pl.BlockSpec((64, 27 * 128), lambda i: (0, 0)),
        ],
        out_specs=[
            pl.BlockSpec((BM, 64), lambda i: (i, 0)),
            pl.BlockSpec((BM, 27 * 128), lambda i: (i, 0)),
        ],
        out_shape=[
            jax.ShapeDtypeStruct((NP, 64), jnp.float32),
            jax.ShapeDtypeStruct((NP, 27 * 128), jnp.float32),
        ],
    )(a2a, a2b, recip, h1, r2, b2, w3f)


def _tck3_body(a3a_ref, a3b_ref, recip_ref, h2_ref, r3_ref, b3_ref,
               wl1_ref, bl1_ref, wl2_ref, bl2_ref, out_ref):
    agg = (a3a_ref[...] + a3b_ref[...]) * recip_ref[...]
    h3 = jax.nn.relu(agg + jnp.dot(h2_ref[...], r3_ref[...],
                                   preferred_element_type=jnp.float32)
                     + b3_ref[...])
    t = jax.nn.relu(jnp.dot(h3, wl1_ref[...],
                            preferred_element_type=jnp.float32) + bl1_ref[...])
    logits = jnp.dot(t.astype(jnp.bfloat16), wl2_ref[...],
                     preferred_element_type=jnp.float32) + bl2_ref[...]
    m = jnp.max(logits, axis=-1, keepdims=True)
    lse = jnp.log(jnp.sum(jnp.exp(logits - m), axis=-1, keepdims=True))
    out_ref[...] = logits - m - lse


@jax.jit
def _tck3(a3a, a3b, recip, h2, r3, b3, wl1, bl1, wl2bf, bl2):
    grid = (NP // BM,)
    return pl.pallas_call(
        _tck3_body,
        grid=grid,
        in_specs=[
            pl.BlockSpec((BM, 128), lambda i: (i, 0)),
            pl.BlockSpec((BM, 128), lambda i: (i, 0)),
            pl.BlockSpec((BM, 1), lambda i: (i, 0)),
            pl.BlockSpec((BM, 64), lambda i: (i, 0)),
            pl.BlockSpec((64, 128), lambda i: (0, 0)),
            pl.BlockSpec((1, 128), lambda i: (0, 0)),
            pl.BlockSpec((128, 256), lambda i: (0, 0)),
            pl.BlockSpec((1, 256), lambda i: (0, 0)),
            pl.BlockSpec((256, NUM_CLASSES), lambda i: (0, 0)),
            pl.BlockSpec((1, NUM_CLASSES), lambda i: (0, 0)),
        ],
        out_specs=pl.BlockSpec((BM, NUM_CLASSES), lambda i: (i, 0)),
        out_shape=jax.ShapeDtypeStruct((NP, NUM_CLASSES), jnp.float32),
    )(a3a, a3b, recip, h2, r3, b3, wl1, bl1, wl2bf, bl2)


def kernel(x, edge_index, pseudo, W1, R1, b1, W2, R2, b2, W3, R3, b3,
           Wl1, bl1, Wl2, bl2):
    pad = EPAD - E
    srcp = jnp.concatenate([edge_index[0], jnp.full((pad,), -1, jnp.int32)])
    dstp = jnp.concatenate([edge_index[1], jnp.zeros((pad,), jnp.int32)])
    psT = jnp.pad(pseudo, ((0, pad), (0, 0))).T.reshape(3 * EPAD)
    xflat = jnp.pad(x.reshape(N), (0, NP - N))
    xp = xflat.reshape(NP, 1)

    w1f = jnp.pad(W1.reshape(27, 32), ((0, 101), (0, 0)))
    w2t = W2.transpose(1, 0, 2)
    w2pad = jnp.concatenate([w2t, jnp.zeros((32, 1, 64), jnp.float32)], 1)
    w2p = jnp.concatenate([w2pad[:, :27], w2pad[:, 1:28]],
                          axis=-1).reshape(32, 27 * 128)
    w3f = W3.transpose(1, 0, 2).reshape(64, 27 * 128)

    w8, gidx, s1 = _k0(srcp, dstp, psT, xflat)
    h1, recip, tab2 = _tck1(s1[0], s1[1], xp, w1f, R1.reshape(1, 32),
                            b1.reshape(1, 32), w2p)
    agg2 = _k23(tab2.reshape(NP * 27, 128), gidx, w8, dstp,
                out_dim=64, paired=True)
    h2, tab3 = _tck2(agg2[0], agg2[1], recip, h1, R2, b2.reshape(1, 64), w3f)
    agg3 = _k23(tab3.reshape(NP * 27, 128), gidx, w8, dstp,
                out_dim=128, paired=False)
    out = _tck3(agg3[0], agg3[1], recip, h2, R3, b3.reshape(1, 128),
                Wl1, bl1.reshape(1, 256), Wl2.astype(jnp.bfloat16),
                bl2.reshape(1, NUM_CLASSES))
    return out[:N]
